# Initial kernel scaffold; baseline (speedup 1.0000x reference)
import jax
import jax.numpy as jnp
from jax.experimental import pallas as pl


def kernel(x_A, edge_index_A, batch_A, x_B, edge_index_B, batch_B, W_in, b_in, W_h1, b_h1, W_h2, b_h2, W_out, b_out):
    raise NotImplementedError("write your pallas kernel here")


# speedup vs baseline: 2.6125x; 2.6125x over previous
import functools

import jax
import jax.numpy as jnp
from jax import lax
from jax.experimental import pallas as pl
from jax.experimental.pallas import tpu as pltpu
from jax.experimental.pallas import tpu_sc as plsc

_N = 10000
_NP = 10240
_E = 320000
_G = 128
_TILES = 16
_C = 80
_BN = 640
_NB = _NP // _BN


def _make_sc_agg_feat():
    mesh = plsc.VectorSubcoreMesh(core_axis_name="c", subcore_axis_name="s")
    epc = _E // _TILES
    nchunks = epc // _C
    rows_pt = _NP // _TILES

    @functools.partial(
        pl.kernel,
        mesh=mesh,
        out_type=jax.ShapeDtypeStruct((2 * _NP, 128), jnp.float32),
        scratch_types=[
            pltpu.VMEM((_C,), jnp.int32),
            pltpu.VMEM((_C,), jnp.int32),
            pltpu.VMEM((_C, 128), jnp.float32),
            pltpu.VMEM_SHARED((_NP, 128), jnp.float32),
            pltpu.SemaphoreType.DMA,
        ],
    )
    def agg_kernel(h_hbm, src2_hbm, dst_hbm, out_hbm, srcb, dstb, rowsb, acc, sem):
        c = lax.axis_index("c")
        s = lax.axis_index("s")
        row0 = s * rows_pt
        pltpu.sync_copy(h_hbm.at[pl.ds(c * _NP + row0, rows_pt)],
                        acc.at[pl.ds(row0, rows_pt)])
        plsc.subcore_barrier()
        ebase = s * epc

        def body(i, carry):
            base = ebase + i * _C
            pltpu.sync_copy(src2_hbm.at[pl.ds(c * _E + base, _C)], srcb)
            pltpu.sync_copy(dst_hbm.at[pl.ds(base, _C)], dstb)
            pltpu.async_copy(h_hbm.at[srcb], rowsb, sem).wait()
            pltpu.sync_copy(rowsb, acc.at[dstb], add=True)
            return carry

        lax.fori_loop(0, nchunks, body, 0)
        plsc.subcore_barrier()
        pltpu.sync_copy(acc.at[pl.ds(row0, rows_pt)],
                        out_hbm.at[pl.ds(c * _NP + row0, rows_pt)])

    return agg_kernel


def _make_sc_agg_edge():
    mesh = plsc.VectorSubcoreMesh(core_axis_name="c", subcore_axis_name="s")
    e_half = _E // 2
    epc = e_half // _TILES
    nchunks = epc // _C
    rows_pt = _NP // _TILES

    @functools.partial(
        pl.kernel,
        mesh=mesh,
        out_type=jax.ShapeDtypeStruct((2 * _NP, 128), jnp.float32),
        scratch_types=[
            pltpu.VMEM((_C,), jnp.int32),
            pltpu.VMEM((_C,), jnp.int32),
            pltpu.VMEM((_C, 128), jnp.float32),
            pltpu.VMEM_SHARED((_NP, 128), jnp.float32),
            pltpu.SemaphoreType.DMA,
        ],
    )
    def agg_kernel(x_hbm, zeros_hbm, src_hbm, dst_hbm, out_hbm,
                   srcb, dstb, rowsb, acc, sem):
        c = lax.axis_index("c")
        s = lax.axis_index("s")
        row0 = s * rows_pt

        @pl.when(c == 0)
        def _():
            pltpu.sync_copy(x_hbm.at[pl.ds(row0, rows_pt)],
                            acc.at[pl.ds(row0, rows_pt)])

        @pl.when(c == 1)
        def _():
            pltpu.sync_copy(zeros_hbm, acc.at[pl.ds(row0, rows_pt)])

        plsc.subcore_barrier()
        ebase = c * e_half + s * epc

        def body(i, carry):
            base = ebase + i * _C
            pltpu.sync_copy(src_hbm.at[pl.ds(base, _C)], srcb)
            pltpu.sync_copy(dst_hbm.at[pl.ds(base, _C)], dstb)
            pltpu.async_copy(x_hbm.at[srcb], rowsb, sem).wait()
            pltpu.sync_copy(rowsb, acc.at[dstb], add=True)
            return carry

        lax.fori_loop(0, nchunks, body, 0)
        plsc.subcore_barrier()
        pltpu.sync_copy(acc.at[pl.ds(row0, rows_pt)],
                        out_hbm.at[pl.ds(c * _NP + row0, rows_pt)])

    return agg_kernel


def _make_tc_layer(mode):
    def body(x_lo_ref, x_hi_ref, w_ref, b_ref, o_ref):
        w = w_ref[...]
        if mode == "partial":
            z = jnp.dot(x_lo_ref[...] + x_hi_ref[...], w,
                        preferred_element_type=jnp.float32)
        else:
            z = (jnp.dot(x_lo_ref[...], w[:128, :],
                         preferred_element_type=jnp.float32)
                 + jnp.dot(x_hi_ref[...], w[128:, :],
                           preferred_element_type=jnp.float32))
        z = z + b_ref[...]
        o_ref[...] = jnp.maximum(z, 0.0)

    w_rows = 128 if mode == "partial" else 256
    return pl.pallas_call(
        body,
        grid=(2, _NB),
        in_specs=[
            pl.BlockSpec((_BN, 128), lambda h, i: (i, 0)),
            pl.BlockSpec((_BN, 128), lambda h, i: (i + _NB, 0)),
            pl.BlockSpec((w_rows, 128), lambda h, i: (0, h)),
            pl.BlockSpec((1, 128), lambda h, i: (0, h)),
        ],
        out_specs=pl.BlockSpec((_BN, 128), lambda h, i: (h * _NB + i, 0)),
        out_shape=jax.ShapeDtypeStruct((2 * _NP, 128), jnp.float32),
    )


def _final_body(hA_lo, hA_hi, bA_ref, hB_lo, hB_hi, bB_ref, w_ref, b_ref,
                o_ref, sumsA, cntA, sumsB, cntB):
    i = pl.program_id(0)
    iota = lax.broadcasted_iota(jnp.int32, (_BN, _G), 1)
    ones_col = jnp.ones((_BN, 1), jnp.float32)
    dn = (((0,), (0,)), ((), ()))
    mA = (bA_ref[...] == iota).astype(jnp.float32)
    mB = (bB_ref[...] == iota).astype(jnp.float32)
    sA_lo = lax.dot_general(mA, hA_lo[...], dn, preferred_element_type=jnp.float32)
    sA_hi = lax.dot_general(mA, hA_hi[...], dn, preferred_element_type=jnp.float32)
    sB_lo = lax.dot_general(mB, hB_lo[...], dn, preferred_element_type=jnp.float32)
    sB_hi = lax.dot_general(mB, hB_hi[...], dn, preferred_element_type=jnp.float32)
    cA = lax.dot_general(mA, ones_col, dn, preferred_element_type=jnp.float32)
    cB = lax.dot_general(mB, ones_col, dn, preferred_element_type=jnp.float32)

    @pl.when(i == 0)
    def _():
        sumsA[:, :128] = sA_lo
        sumsA[:, 128:] = sA_hi
        sumsB[:, :128] = sB_lo
        sumsB[:, 128:] = sB_hi
        cntA[...] = cA
        cntB[...] = cB

    @pl.when(i > 0)
    def _():
        sumsA[:, :128] += sA_lo
        sumsA[:, 128:] += sA_hi
        sumsB[:, :128] += sB_lo
        sumsB[:, 128:] += sB_hi
        cntA[...] += cA
        cntB[...] += cB

    @pl.when(i == _NB - 1)
    def _():
        pooledA = sumsA[...] / jnp.maximum(cntA[...], 1.0)
        pooledB = sumsB[...] / jnp.maximum(cntB[...], 1.0)
        w = w_ref[...]
        b = b_ref[...]
        embA = jnp.dot(pooledA, w, preferred_element_type=jnp.float32) + b
        embB = jnp.dot(pooledB, w, preferred_element_type=jnp.float32) + b
        num = jnp.sum(embA * embB, axis=1, keepdims=True)
        nA = jnp.sqrt(jnp.sum(embA * embA, axis=1, keepdims=True))
        nB = jnp.sqrt(jnp.sum(embB * embB, axis=1, keepdims=True))
        o_ref[...] = num / jnp.maximum(nA * nB, 1e-8)


def _make_final():
    return pl.pallas_call(
        _final_body,
        grid=(_NB,),
        in_specs=[
            pl.BlockSpec((_BN, 128), lambda i: (i, 0)),
            pl.BlockSpec((_BN, 128), lambda i: (i + _NB, 0)),
            pl.BlockSpec((_BN, 1), lambda i: (i, 0)),
            pl.BlockSpec((_BN, 128), lambda i: (i, 0)),
            pl.BlockSpec((_BN, 128), lambda i: (i + _NB, 0)),
            pl.BlockSpec((_BN, 1), lambda i: (i, 0)),
            pl.BlockSpec((256, 128), lambda i: (0, 0)),
            pl.BlockSpec((1, 128), lambda i: (0, 0)),
        ],
        out_specs=pl.BlockSpec((_G, 1), lambda i: (0, 0)),
        out_shape=jax.ShapeDtypeStruct((_G, 1), jnp.float32),
        scratch_shapes=[
            pltpu.VMEM((_G, 256), jnp.float32),
            pltpu.VMEM((_G, 1), jnp.float32),
            pltpu.VMEM((_G, 256), jnp.float32),
            pltpu.VMEM((_G, 1), jnp.float32),
        ],
    )


def kernel(x_A, edge_index_A, batch_A, x_B, edge_index_B, batch_B,
           W_in, b_in, W_h1, b_h1, W_h2, b_h2, W_out, b_out):
    agg_edge = _make_sc_agg_edge()
    agg_feat = _make_sc_agg_feat()
    layer1 = _make_tc_layer("partial")
    layer23 = _make_tc_layer("halves")
    final = _make_final()

    b_in2 = b_in.astype(jnp.float32).reshape(1, 256)
    b_h12 = b_h1.astype(jnp.float32).reshape(1, 256)
    b_h22 = b_h2.astype(jnp.float32).reshape(1, 256)
    b_out2 = b_out.astype(jnp.float32).reshape(1, 128)
    pad = jnp.zeros((_NP - _N, 128), jnp.float32)
    zeros_pt = jnp.zeros((_NP // _TILES, 128), jnp.float32)

    def gnn(x, edge_index):
        src = edge_index[0].astype(jnp.int32)
        dst = edge_index[1].astype(jnp.int32)
        src2 = jnp.concatenate([src, src + _NP])
        x_p = jnp.concatenate([x, pad])
        a1 = agg_edge(x_p, zeros_pt, src, dst)
        h1 = layer1(a1, a1, W_in, b_in2)
        a2 = agg_feat(h1, src2, dst)
        h2 = layer23(a2, a2, W_h1, b_h12)
        a3 = agg_feat(h2, src2, dst)
        h3 = layer23(a3, a3, W_h2, b_h22)
        return h3

    def pad_batch(batch):
        b = jnp.concatenate([batch.astype(jnp.int32),
                             jnp.full((_NP - _N,), -1, jnp.int32)])
        return b.reshape(_NP, 1)

    h3A = gnn(x_A, edge_index_A)
    h3B = gnn(x_B, edge_index_B)
    score = final(h3A, h3A, pad_batch(batch_A),
                  h3B, h3B, pad_batch(batch_B),
                  W_out, b_out2)
    return score.reshape(_G)


# speedup vs baseline: 4.6789x; 1.7910x over previous
import functools

import jax
import jax.numpy as jnp
from jax import lax
from jax.experimental import pallas as pl
from jax.experimental.pallas import tpu as pltpu
from jax.experimental.pallas import tpu_sc as plsc

_N = 10000
_NP = 10240
_E = 320000
_G = 128
_TILES = 16
_BN = 640
_NB = _NP // _BN


def _edge_pipeline(table_hbm, src_hbm, src_base, dst_hbm, dst_base, acc,
                   srcall, dstb0, dstb1, rowsb0, rowsb1,
                   g0, g1, s0, s1, d0, d1, C, epc):
    nmac = (epc // C) // 2
    pltpu.sync_copy(src_hbm.at[pl.ds(src_base, epc)], srcall)

    def src_sl(k):
        return srcall.at[pl.ds(k * C, C)]

    def g_start(k, rb, sg):
        pltpu.async_copy(table_hbm.at[src_sl(k)], rb, sg)

    def g_wait(k, rb, sg):
        pltpu.make_async_copy(table_hbm.at[src_sl(k)], rb, sg).wait()

    def d_start(k, db, sd):
        pltpu.async_copy(dst_hbm.at[pl.ds(dst_base + k * C, C)], db, sd)

    def d_wait(k, db, sd):
        pltpu.make_async_copy(dst_hbm.at[pl.ds(dst_base + k * C, C)], db, sd).wait()

    def s_start(db, rb, ss):
        pltpu.async_copy(rb, acc.at[db], ss, add=True)

    def s_wait(db, rb, ss):
        pltpu.make_async_copy(rb, acc.at[db], ss).wait()

    d_start(0, dstb0, d0)
    g_start(0, rowsb0, g0)

    def body(m, carry):
        a = 2 * m
        g_wait(a, rowsb0, g0)
        d_wait(a, dstb0, d0)
        s_start(dstb0, rowsb0, s0)

        @pl.when(m > 0)
        def _():
            s_wait(dstb1, rowsb1, s1)

        d_start(a + 1, dstb1, d1)
        g_start(a + 1, rowsb1, g1)
        s_wait(dstb0, rowsb0, s0)

        @pl.when(m < nmac - 1)
        def _():
            d_start(a + 2, dstb0, d0)

        g_wait(a + 1, rowsb1, g1)
        d_wait(a + 1, dstb1, d1)
        s_start(dstb1, rowsb1, s1)

        @pl.when(m < nmac - 1)
        def _():
            g_start(a + 2, rowsb0, g0)

        return carry

    lax.fori_loop(0, nmac, body, 0)
    s_wait(dstb1, rowsb1, s1)


def _sc_scratch(C, epc):
    return [
        pltpu.VMEM((epc,), jnp.int32),
        pltpu.VMEM((C,), jnp.int32),
        pltpu.VMEM((C,), jnp.int32),
        pltpu.VMEM((C, 128), jnp.float32),
        pltpu.VMEM((C, 128), jnp.float32),
        pltpu.VMEM_SHARED((_NP, 128), jnp.float32),
        pltpu.SemaphoreType.DMA,
        pltpu.SemaphoreType.DMA,
        pltpu.SemaphoreType.DMA,
        pltpu.SemaphoreType.DMA,
        pltpu.SemaphoreType.DMA,
        pltpu.SemaphoreType.DMA,
    ]


def _make_sc_agg_feat():
    mesh = plsc.VectorSubcoreMesh(core_axis_name="c", subcore_axis_name="s")
    C = 80
    epc = _E // _TILES
    rows_pt = _NP // _TILES

    @functools.partial(
        pl.kernel,
        mesh=mesh,
        out_type=jax.ShapeDtypeStruct((2 * _NP, 128), jnp.float32),
        scratch_types=_sc_scratch(C, epc),
    )
    def agg_kernel(h_hbm, src2_hbm, dst_hbm, out_hbm,
                   srcall, dstb0, dstb1, rowsb0, rowsb1, acc,
                   g0, g1, s0, s1, d0, d1):
        c = lax.axis_index("c")
        s = lax.axis_index("s")
        row0 = s * rows_pt
        pltpu.sync_copy(h_hbm.at[pl.ds(c * _NP + row0, rows_pt)],
                        acc.at[pl.ds(row0, rows_pt)])
        plsc.subcore_barrier()
        _edge_pipeline(h_hbm, src2_hbm, c * _E + s * epc, dst_hbm, s * epc,
                       acc, srcall, dstb0, dstb1,
                       rowsb0, rowsb1, g0, g1, s0, s1, d0, d1, C, epc)
        plsc.subcore_barrier()
        pltpu.sync_copy(acc.at[pl.ds(row0, rows_pt)],
                        out_hbm.at[pl.ds(c * _NP + row0, rows_pt)])

    return agg_kernel


def _make_sc_agg_edge():
    mesh = plsc.VectorSubcoreMesh(core_axis_name="c", subcore_axis_name="s")
    C = 40
    e_half = _E // 2
    epc = e_half // _TILES
    rows_pt = _NP // _TILES

    @functools.partial(
        pl.kernel,
        mesh=mesh,
        out_type=jax.ShapeDtypeStruct((2 * _NP, 128), jnp.float32),
        scratch_types=_sc_scratch(C, epc),
    )
    def agg_kernel(x_hbm, zeros_hbm, src_hbm, dst_hbm, out_hbm,
                   srcall, dstb0, dstb1, rowsb0, rowsb1, acc,
                   g0, g1, s0, s1, d0, d1):
        c = lax.axis_index("c")
        s = lax.axis_index("s")
        row0 = s * rows_pt

        @pl.when(c == 0)
        def _():
            pltpu.sync_copy(x_hbm.at[pl.ds(row0, rows_pt)],
                            acc.at[pl.ds(row0, rows_pt)])

        @pl.when(c == 1)
        def _():
            pltpu.sync_copy(zeros_hbm, acc.at[pl.ds(row0, rows_pt)])

        plsc.subcore_barrier()
        base = c * e_half + s * epc
        _edge_pipeline(x_hbm, src_hbm, base, dst_hbm, base,
                       acc, srcall, dstb0, dstb1,
                       rowsb0, rowsb1, g0, g1, s0, s1, d0, d1, C, epc)
        plsc.subcore_barrier()
        pltpu.sync_copy(acc.at[pl.ds(row0, rows_pt)],
                        out_hbm.at[pl.ds(c * _NP + row0, rows_pt)])

    return agg_kernel


def _make_tc_layer(mode):
    def body(x_lo_ref, x_hi_ref, w_ref, b_ref, o_ref):
        w = w_ref[...]
        if mode == "partial":
            z = jnp.dot(x_lo_ref[...] + x_hi_ref[...], w,
                        preferred_element_type=jnp.float32)
        else:
            z = (jnp.dot(x_lo_ref[...], w[:128, :],
                         preferred_element_type=jnp.float32)
                 + jnp.dot(x_hi_ref[...], w[128:, :],
                           preferred_element_type=jnp.float32))
        z = z + b_ref[...]
        o_ref[...] = jnp.maximum(z, 0.0)

    w_rows = 128 if mode == "partial" else 256
    return pl.pallas_call(
        body,
        grid=(2, _NB),
        in_specs=[
            pl.BlockSpec((_BN, 128), lambda h, i: (i, 0)),
            pl.BlockSpec((_BN, 128), lambda h, i: (i + _NB, 0)),
            pl.BlockSpec((w_rows, 128), lambda h, i: (0, h)),
            pl.BlockSpec((1, 128), lambda h, i: (0, h)),
        ],
        out_specs=pl.BlockSpec((_BN, 128), lambda h, i: (h * _NB + i, 0)),
        out_shape=jax.ShapeDtypeStruct((2 * _NP, 128), jnp.float32),
    )


def _final_body(hA_lo, hA_hi, bA_ref, hB_lo, hB_hi, bB_ref, w_ref, b_ref,
                o_ref, sumsA, cntA, sumsB, cntB):
    i = pl.program_id(0)
    iota = lax.broadcasted_iota(jnp.int32, (_BN, _G), 1)
    ones_col = jnp.ones((_BN, 1), jnp.float32)
    dn = (((0,), (0,)), ((), ()))
    mA = (bA_ref[...] == iota).astype(jnp.float32)
    mB = (bB_ref[...] == iota).astype(jnp.float32)
    sA_lo = lax.dot_general(mA, hA_lo[...], dn, preferred_element_type=jnp.float32)
    sA_hi = lax.dot_general(mA, hA_hi[...], dn, preferred_element_type=jnp.float32)
    sB_lo = lax.dot_general(mB, hB_lo[...], dn, preferred_element_type=jnp.float32)
    sB_hi = lax.dot_general(mB, hB_hi[...], dn, preferred_element_type=jnp.float32)
    cA = lax.dot_general(mA, ones_col, dn, preferred_element_type=jnp.float32)
    cB = lax.dot_general(mB, ones_col, dn, preferred_element_type=jnp.float32)

    @pl.when(i == 0)
    def _():
        sumsA[:, :128] = sA_lo
        sumsA[:, 128:] = sA_hi
        sumsB[:, :128] = sB_lo
        sumsB[:, 128:] = sB_hi
        cntA[...] = cA
        cntB[...] = cB

    @pl.when(i > 0)
    def _():
        sumsA[:, :128] += sA_lo
        sumsA[:, 128:] += sA_hi
        sumsB[:, :128] += sB_lo
        sumsB[:, 128:] += sB_hi
        cntA[...] += cA
        cntB[...] += cB

    @pl.when(i == _NB - 1)
    def _():
        pooledA = sumsA[...] / jnp.maximum(cntA[...], 1.0)
        pooledB = sumsB[...] / jnp.maximum(cntB[...], 1.0)
        w = w_ref[...]
        b = b_ref[...]
        embA = jnp.dot(pooledA, w, preferred_element_type=jnp.float32) + b
        embB = jnp.dot(pooledB, w, preferred_element_type=jnp.float32) + b
        num = jnp.sum(embA * embB, axis=1, keepdims=True)
        nA = jnp.sqrt(jnp.sum(embA * embA, axis=1, keepdims=True))
        nB = jnp.sqrt(jnp.sum(embB * embB, axis=1, keepdims=True))
        o_ref[...] = num / jnp.maximum(nA * nB, 1e-8)


def _make_final():
    return pl.pallas_call(
        _final_body,
        grid=(_NB,),
        in_specs=[
            pl.BlockSpec((_BN, 128), lambda i: (i, 0)),
            pl.BlockSpec((_BN, 128), lambda i: (i + _NB, 0)),
            pl.BlockSpec((_BN, 1), lambda i: (i, 0)),
            pl.BlockSpec((_BN, 128), lambda i: (i, 0)),
            pl.BlockSpec((_BN, 128), lambda i: (i + _NB, 0)),
            pl.BlockSpec((_BN, 1), lambda i: (i, 0)),
            pl.BlockSpec((256, 128), lambda i: (0, 0)),
            pl.BlockSpec((1, 128), lambda i: (0, 0)),
        ],
        out_specs=pl.BlockSpec((_G, 1), lambda i: (0, 0)),
        out_shape=jax.ShapeDtypeStruct((_G, 1), jnp.float32),
        scratch_shapes=[
            pltpu.VMEM((_G, 256), jnp.float32),
            pltpu.VMEM((_G, 1), jnp.float32),
            pltpu.VMEM((_G, 256), jnp.float32),
            pltpu.VMEM((_G, 1), jnp.float32),
        ],
    )


def kernel(x_A, edge_index_A, batch_A, x_B, edge_index_B, batch_B,
           W_in, b_in, W_h1, b_h1, W_h2, b_h2, W_out, b_out):
    agg_edge = _make_sc_agg_edge()
    agg_feat = _make_sc_agg_feat()
    layer1 = _make_tc_layer("partial")
    layer23 = _make_tc_layer("halves")
    final = _make_final()

    b_in2 = b_in.astype(jnp.float32).reshape(1, 256)
    b_h12 = b_h1.astype(jnp.float32).reshape(1, 256)
    b_h22 = b_h2.astype(jnp.float32).reshape(1, 256)
    b_out2 = b_out.astype(jnp.float32).reshape(1, 128)
    pad = jnp.zeros((_NP - _N, 128), jnp.float32)
    zeros_pt = jnp.zeros((_NP // _TILES, 128), jnp.float32)

    def gnn(x, edge_index):
        src = edge_index[0].astype(jnp.int32)
        dst = edge_index[1].astype(jnp.int32)
        src2 = jnp.concatenate([src, src + _NP])
        x_p = jnp.concatenate([x, pad])
        a1 = agg_edge(x_p, zeros_pt, src, dst)
        h1 = layer1(a1, a1, W_in, b_in2)
        a2 = agg_feat(h1, src2, dst)
        h2 = layer23(a2, a2, W_h1, b_h12)
        a3 = agg_feat(h2, src2, dst)
        h3 = layer23(a3, a3, W_h2, b_h22)
        return h3

    def pad_batch(batch):
        b = jnp.concatenate([batch.astype(jnp.int32),
                             jnp.full((_NP - _N,), -1, jnp.int32)])
        return b.reshape(_NP, 1)

    h3A = gnn(x_A, edge_index_A)
    h3B = gnn(x_B, edge_index_B)
    score = final(h3A, h3A, pad_batch(batch_A),
                  h3B, h3B, pad_batch(batch_B),
                  W_out, b_out2)
    return score.reshape(_G)


# speedup vs baseline: 8.1107x; 1.7334x over previous
import functools

import jax
import jax.numpy as jnp
from jax import lax
from jax.experimental import pallas as pl
from jax.experimental.pallas import tpu as pltpu
from jax.experimental.pallas import tpu_sc as plsc

_N = 10000
_NP = 10240
_E = 320000
_G = 128
_TILES = 16
_BN = 640
_NB = _NP // _BN


_R = 5


def _edge_pipeline(table_hbm, src_hbm, src_base, dst_hbm, dst_base, acc,
                   srcall, dstbs, rowsbs, gs, ss, ds, C, epc):
    n = epc // C
    nmac = n // _R
    pltpu.sync_copy(src_hbm.at[pl.ds(src_base, epc)], srcall)

    def src_sl(k):
        return srcall.at[pl.ds(k * C, C)]

    def g_start(k, p):
        pltpu.async_copy(table_hbm.at[src_sl(k)], rowsbs[p], gs[p])

    def g_wait(k, p):
        pltpu.make_async_copy(table_hbm.at[src_sl(k)], rowsbs[p], gs[p]).wait()

    def d_start(k, p):
        pltpu.async_copy(dst_hbm.at[pl.ds(dst_base + k * C, C)], dstbs[p], ds[p])

    def d_wait(k, p):
        pltpu.make_async_copy(dst_hbm.at[pl.ds(dst_base + k * C, C)],
                              dstbs[p], ds[p]).wait()

    def s_start(p):
        pltpu.async_copy(rowsbs[p], acc.at[dstbs[p]], ss[p], add=True)

    def s_wait(p):
        pltpu.make_async_copy(rowsbs[p], acc.at[dstbs[p]], ss[p]).wait()

    for j in range(_R - 1):
        d_start(j, j)
        g_start(j, j)

    def body(m, carry):
        k0 = m * _R
        for t in range(_R):
            k = k0 + t
            q = (t + _R - 1) % _R
            g_wait(k, t)
            d_wait(k, t)
            s_start(t)
            if t == 0:
                @pl.when(m > 0)
                def _():
                    s_wait(q)
            else:
                s_wait(q)

            @pl.when(k + _R - 1 < n)
            def _():
                d_start(k + _R - 1, q)
                g_start(k + _R - 1, q)

        return carry

    lax.fori_loop(0, nmac, body, 0)
    s_wait((_R - 1) % _R)


def _sc_scratch(C, epc):
    return ([pltpu.VMEM((epc,), jnp.int32)]
            + [pltpu.VMEM((C,), jnp.int32) for _ in range(_R)]
            + [pltpu.VMEM((C, 128), jnp.float32) for _ in range(_R)]
            + [pltpu.VMEM_SHARED((_NP, 128), jnp.float32)]
            + [pltpu.SemaphoreType.DMA for _ in range(3 * _R)])


def _make_sc_agg_feat():
    mesh = plsc.VectorSubcoreMesh(core_axis_name="c", subcore_axis_name="s")
    C = 40
    epc = _E // _TILES
    rows_pt = _NP // _TILES

    @functools.partial(
        pl.kernel,
        mesh=mesh,
        out_type=jax.ShapeDtypeStruct((2 * _NP, 128), jnp.float32),
        scratch_types=_sc_scratch(C, epc),
    )
    def agg_kernel(h_hbm, src2_hbm, dst_hbm, out_hbm, srcall, *scr):
        dstbs = scr[:_R]
        rowsbs = scr[_R:2 * _R]
        acc = scr[2 * _R]
        gs = scr[2 * _R + 1:3 * _R + 1]
        ss = scr[3 * _R + 1:4 * _R + 1]
        ds = scr[4 * _R + 1:5 * _R + 1]
        c = lax.axis_index("c")
        s = lax.axis_index("s")
        row0 = s * rows_pt
        pltpu.sync_copy(h_hbm.at[pl.ds(c * _NP + row0, rows_pt)],
                        acc.at[pl.ds(row0, rows_pt)])
        plsc.subcore_barrier()
        _edge_pipeline(h_hbm, src2_hbm, c * _E + s * epc, dst_hbm, s * epc,
                       acc, srcall, dstbs, rowsbs, gs, ss, ds, C, epc)
        plsc.subcore_barrier()
        pltpu.sync_copy(acc.at[pl.ds(row0, rows_pt)],
                        out_hbm.at[pl.ds(c * _NP + row0, rows_pt)])

    return agg_kernel


def _make_sc_agg_edge():
    mesh = plsc.VectorSubcoreMesh(core_axis_name="c", subcore_axis_name="s")
    C = 40
    e_half = _E // 2
    epc = e_half // _TILES
    rows_pt = _NP // _TILES

    @functools.partial(
        pl.kernel,
        mesh=mesh,
        out_type=jax.ShapeDtypeStruct((2 * _NP, 128), jnp.float32),
        scratch_types=_sc_scratch(C, epc),
    )
    def agg_kernel(x_hbm, zeros_hbm, src_hbm, dst_hbm, out_hbm, srcall, *scr):
        dstbs = scr[:_R]
        rowsbs = scr[_R:2 * _R]
        acc = scr[2 * _R]
        gs = scr[2 * _R + 1:3 * _R + 1]
        ss = scr[3 * _R + 1:4 * _R + 1]
        ds = scr[4 * _R + 1:5 * _R + 1]
        c = lax.axis_index("c")
        s = lax.axis_index("s")
        row0 = s * rows_pt

        @pl.when(c == 0)
        def _():
            pltpu.sync_copy(x_hbm.at[pl.ds(row0, rows_pt)],
                            acc.at[pl.ds(row0, rows_pt)])

        @pl.when(c == 1)
        def _():
            pltpu.sync_copy(zeros_hbm, acc.at[pl.ds(row0, rows_pt)])

        plsc.subcore_barrier()
        base = c * e_half + s * epc
        _edge_pipeline(x_hbm, src_hbm, base, dst_hbm, base,
                       acc, srcall, dstbs, rowsbs, gs, ss, ds, C, epc)
        plsc.subcore_barrier()
        pltpu.sync_copy(acc.at[pl.ds(row0, rows_pt)],
                        out_hbm.at[pl.ds(c * _NP + row0, rows_pt)])

    return agg_kernel


def _make_tc_layer(mode):
    def body(x_lo_ref, x_hi_ref, w_ref, b_ref, o_ref):
        w = w_ref[...]
        if mode == "partial":
            z = jnp.dot(x_lo_ref[...] + x_hi_ref[...], w,
                        preferred_element_type=jnp.float32)
        else:
            z = (jnp.dot(x_lo_ref[...], w[:128, :],
                         preferred_element_type=jnp.float32)
                 + jnp.dot(x_hi_ref[...], w[128:, :],
                           preferred_element_type=jnp.float32))
        z = z + b_ref[...]
        o_ref[...] = jnp.maximum(z, 0.0)

    w_rows = 128 if mode == "partial" else 256
    return pl.pallas_call(
        body,
        grid=(2, _NB),
        in_specs=[
            pl.BlockSpec((_BN, 128), lambda h, i: (i, 0)),
            pl.BlockSpec((_BN, 128), lambda h, i: (i + _NB, 0)),
            pl.BlockSpec((w_rows, 128), lambda h, i: (0, h)),
            pl.BlockSpec((1, 128), lambda h, i: (0, h)),
        ],
        out_specs=pl.BlockSpec((_BN, 128), lambda h, i: (h * _NB + i, 0)),
        out_shape=jax.ShapeDtypeStruct((2 * _NP, 128), jnp.float32),
    )


def _final_body(hA_lo, hA_hi, bA_ref, hB_lo, hB_hi, bB_ref, w_ref, b_ref,
                o_ref, sumsA, cntA, sumsB, cntB):
    i = pl.program_id(0)
    iota = lax.broadcasted_iota(jnp.int32, (_BN, _G), 1)
    ones_col = jnp.ones((_BN, 1), jnp.float32)
    dn = (((0,), (0,)), ((), ()))
    mA = (bA_ref[...] == iota).astype(jnp.float32)
    mB = (bB_ref[...] == iota).astype(jnp.float32)
    sA_lo = lax.dot_general(mA, hA_lo[...], dn, preferred_element_type=jnp.float32)
    sA_hi = lax.dot_general(mA, hA_hi[...], dn, preferred_element_type=jnp.float32)
    sB_lo = lax.dot_general(mB, hB_lo[...], dn, preferred_element_type=jnp.float32)
    sB_hi = lax.dot_general(mB, hB_hi[...], dn, preferred_element_type=jnp.float32)
    cA = lax.dot_general(mA, ones_col, dn, preferred_element_type=jnp.float32)
    cB = lax.dot_general(mB, ones_col, dn, preferred_element_type=jnp.float32)

    @pl.when(i == 0)
    def _():
        sumsA[:, :128] = sA_lo
        sumsA[:, 128:] = sA_hi
        sumsB[:, :128] = sB_lo
        sumsB[:, 128:] = sB_hi
        cntA[...] = cA
        cntB[...] = cB

    @pl.when(i > 0)
    def _():
        sumsA[:, :128] += sA_lo
        sumsA[:, 128:] += sA_hi
        sumsB[:, :128] += sB_lo
        sumsB[:, 128:] += sB_hi
        cntA[...] += cA
        cntB[...] += cB

    @pl.when(i == _NB - 1)
    def _():
        pooledA = sumsA[...] / jnp.maximum(cntA[...], 1.0)
        pooledB = sumsB[...] / jnp.maximum(cntB[...], 1.0)
        w = w_ref[...]
        b = b_ref[...]
        embA = jnp.dot(pooledA, w, preferred_element_type=jnp.float32) + b
        embB = jnp.dot(pooledB, w, preferred_element_type=jnp.float32) + b
        num = jnp.sum(embA * embB, axis=1, keepdims=True)
        nA = jnp.sqrt(jnp.sum(embA * embA, axis=1, keepdims=True))
        nB = jnp.sqrt(jnp.sum(embB * embB, axis=1, keepdims=True))
        o_ref[...] = num / jnp.maximum(nA * nB, 1e-8)


def _make_final():
    return pl.pallas_call(
        _final_body,
        grid=(_NB,),
        in_specs=[
            pl.BlockSpec((_BN, 128), lambda i: (i, 0)),
            pl.BlockSpec((_BN, 128), lambda i: (i + _NB, 0)),
            pl.BlockSpec((_BN, 1), lambda i: (i, 0)),
            pl.BlockSpec((_BN, 128), lambda i: (i, 0)),
            pl.BlockSpec((_BN, 128), lambda i: (i + _NB, 0)),
            pl.BlockSpec((_BN, 1), lambda i: (i, 0)),
            pl.BlockSpec((256, 128), lambda i: (0, 0)),
            pl.BlockSpec((1, 128), lambda i: (0, 0)),
        ],
        out_specs=pl.BlockSpec((_G, 1), lambda i: (0, 0)),
        out_shape=jax.ShapeDtypeStruct((_G, 1), jnp.float32),
        scratch_shapes=[
            pltpu.VMEM((_G, 256), jnp.float32),
            pltpu.VMEM((_G, 1), jnp.float32),
            pltpu.VMEM((_G, 256), jnp.float32),
            pltpu.VMEM((_G, 1), jnp.float32),
        ],
    )


def kernel(x_A, edge_index_A, batch_A, x_B, edge_index_B, batch_B,
           W_in, b_in, W_h1, b_h1, W_h2, b_h2, W_out, b_out):
    agg_edge = _make_sc_agg_edge()
    agg_feat = _make_sc_agg_feat()
    layer1 = _make_tc_layer("partial")
    layer23 = _make_tc_layer("halves")
    final = _make_final()

    b_in2 = b_in.astype(jnp.float32).reshape(1, 256)
    b_h12 = b_h1.astype(jnp.float32).reshape(1, 256)
    b_h22 = b_h2.astype(jnp.float32).reshape(1, 256)
    b_out2 = b_out.astype(jnp.float32).reshape(1, 128)
    pad = jnp.zeros((_NP - _N, 128), jnp.float32)
    zeros_pt = jnp.zeros((_NP // _TILES, 128), jnp.float32)

    def gnn(x, edge_index):
        src = edge_index[0].astype(jnp.int32)
        dst = edge_index[1].astype(jnp.int32)
        src2 = jnp.concatenate([src, src + _NP])
        x_p = jnp.concatenate([x, pad])
        a1 = agg_edge(x_p, zeros_pt, src, dst)
        h1 = layer1(a1, a1, W_in, b_in2)
        a2 = agg_feat(h1, src2, dst)
        h2 = layer23(a2, a2, W_h1, b_h12)
        a3 = agg_feat(h2, src2, dst)
        h3 = layer23(a3, a3, W_h2, b_h22)
        return h3

    def pad_batch(batch):
        b = jnp.concatenate([batch.astype(jnp.int32),
                             jnp.full((_NP - _N,), -1, jnp.int32)])
        return b.reshape(_NP, 1)

    h3A = gnn(x_A, edge_index_A)
    h3B = gnn(x_B, edge_index_B)
    score = final(h3A, h3A, pad_batch(batch_A),
                  h3B, h3B, pad_batch(batch_B),
                  W_out, b_out2)
    return score.reshape(_G)
